# Initial kernel scaffold; baseline (speedup 1.0000x reference)
import jax
import jax.numpy as jnp
from jax.experimental import pallas as pl


def kernel(x, edge_index, edge_values, W):
    raise NotImplementedError("write your pallas kernel here")


# speedup vs baseline: 3.5451x; 3.5451x over previous
import functools

import jax
import jax.numpy as jnp
from jax import lax
from jax.experimental import pallas as pl
from jax.experimental.pallas import tpu as pltpu
from jax.experimental.pallas import tpu_sc as plsc

N = 10000
E = 320000
D = 128

NC = 2
NS = 16
L = 16

NW = NC * NS
CH = 128
K = 80
E_PAD = NW * K * CH
K2 = K // 2

ROWS_PT = N // NS
OUT_PT = (N // NS) // 8 * 8
OUT_TAIL = N - OUT_PT * NS


def _combine_matmul_body(p_ref, w_ref, o_ref):
    o_ref[...] = lax.dot_general(
        p_ref[0] + p_ref[1], w_ref[...], (((1,), (1,)), ((), ())),
        preferred_element_type=jnp.float32)


def _tc_combine_matmul(partials, w):
    bm = 2000
    return pl.pallas_call(
        _combine_matmul_body,
        grid=(N // bm,),
        in_specs=[
            pl.BlockSpec((NC, bm, D), lambda i: (0, i, 0)),
            pl.BlockSpec((D, D), lambda i: (0, 0)),
        ],
        out_specs=pl.BlockSpec((bm, D), lambda i: (i, 0)),
        out_shape=jax.ShapeDtypeStruct((N, D), jnp.float32),
    )(partials, w)


def _sc_body(x_hbm, src_hbm, dst_hbm, val_hbm, out_hbm,
             src_all, dst_all, val_all, rows_a, rows_b, acc_sh,
             sem_ga, sem_gb, sem_sa, sem_sb):
    c = lax.axis_index("c")
    s = lax.axis_index("s")
    wid = c * NS + s

    zero = jnp.zeros((L,), jnp.float32)

    def z_body(i, carry):
        for r in range(D // L):
            rows_a[i, pl.ds(r * L, L)] = zero
        return carry
    lax.fori_loop(0, CH, z_body, 0)

    rbase = s * ROWS_PT
    for k in range(ROWS_PT // CH):
        pltpu.sync_copy(rows_a, acc_sh.at[pl.ds(rbase + k * CH, CH)])
    tail = ROWS_PT - (ROWS_PT // CH) * CH
    if tail:
        pltpu.sync_copy(rows_a.at[pl.ds(0, tail)],
                        acc_sh.at[pl.ds(rbase + (ROWS_PT // CH) * CH, tail)])
    plsc.subcore_barrier()

    def g_start(chunk, rows_ref, sem):
        pltpu.async_copy(x_hbm.at[src_all.at[chunk]], rows_ref, sem)

    def g_wait(rows_ref, sem):
        pltpu.make_async_copy(x_hbm.at[src_all.at[0]], rows_ref, sem).wait()

    def s_start(chunk, rows_ref, sem):
        pltpu.async_copy(rows_ref, acc_sh.at[dst_all.at[chunk]], sem, add=True)

    def s_wait(rows_ref, sem):
        pltpu.make_async_copy(rows_ref, acc_sh.at[dst_all.at[0]], sem).wait()

    def scale(rows_ref, chunk):
        def g_body(g, carry):
            vals = val_all[chunk, pl.ds(g * L, L)]
            for i in range(L):
                v = vals[i]
                e = g * L + i
                for r in range(D // L):
                    sl = pl.ds(r * L, L)
                    rows_ref[e, sl] = rows_ref[e, sl] * v
            return carry
        lax.fori_loop(0, CH // L, g_body, 0)

    P = K2 // 2
    for stage in range(K // K2):
        sbase = stage * K2
        pltpu.sync_copy(src_hbm.at[wid, pl.ds(sbase, K2)], src_all)
        pltpu.sync_copy(dst_hbm.at[wid, pl.ds(sbase, K2)], dst_all)
        pltpu.sync_copy(val_hbm.at[wid, pl.ds(sbase, K2)], val_all)

        g_start(0, rows_a, sem_ga)
        g_start(1, rows_b, sem_gb)

        def pair(p, carry):
            ca = 2 * p
            cb = ca + 1

            g_wait(rows_a, sem_ga)
            scale(rows_a, ca)
            s_start(ca, rows_a, sem_sa)

            @pl.when(p < P - 1)
            def _refill_a():
                s_wait(rows_a, sem_sa)
                g_start(ca + 2, rows_a, sem_ga)

            g_wait(rows_b, sem_gb)
            scale(rows_b, cb)
            s_start(cb, rows_b, sem_sb)

            @pl.when(p < P - 1)
            def _refill_b():
                s_wait(rows_b, sem_sb)
                g_start(cb + 2, rows_b, sem_gb)
            return carry
        lax.fori_loop(0, P, pair, 0)
        s_wait(rows_a, sem_sa)
        s_wait(rows_b, sem_sb)

    plsc.subcore_barrier()
    rows0 = s * OUT_PT
    pltpu.sync_copy(acc_sh.at[pl.ds(rows0, OUT_PT)],
                    out_hbm.at[c, pl.ds(rows0, OUT_PT)])

    @pl.when(s == 0)
    def _copy_tail():
        pltpu.sync_copy(acc_sh.at[pl.ds(OUT_PT * NS, OUT_TAIL)],
                        out_hbm.at[c, pl.ds(OUT_PT * NS, OUT_TAIL)])


_sc_aggregate = functools.partial(
    pl.kernel,
    out_type=jax.ShapeDtypeStruct((NC, N, D), jnp.float32),
    mesh=plsc.VectorSubcoreMesh(core_axis_name="c", subcore_axis_name="s"),
    scratch_types=[
        pltpu.VMEM((K2, CH), jnp.int32),
        pltpu.VMEM((K2, CH), jnp.int32),
        pltpu.VMEM((K2, CH), jnp.float32),
        pltpu.VMEM((CH, D), jnp.float32),
        pltpu.VMEM((CH, D), jnp.float32),
        pltpu.VMEM_SHARED((N, D), jnp.float32),
        pltpu.SemaphoreType.DMA,
        pltpu.SemaphoreType.DMA,
        pltpu.SemaphoreType.DMA,
        pltpu.SemaphoreType.DMA,
    ],
)(_sc_body)


def kernel(x, edge_index, edge_values, W):
    dst = edge_index[0].astype(jnp.int32)
    src = edge_index[1].astype(jnp.int32)
    pad = E_PAD - E
    src3 = jnp.pad(src, (0, pad)).reshape(NW, K, CH)
    dst3 = jnp.pad(dst, (0, pad)).reshape(NW, K, CH)
    val3 = jnp.pad(edge_values, (0, pad)).reshape(NW, K, CH)
    partials = _sc_aggregate(x, src3, dst3, val3)
    return _tc_combine_matmul(partials, W)


# speedup vs baseline: 11.4783x; 3.2378x over previous
import functools

import jax
import jax.numpy as jnp
from jax import lax
from jax.experimental import pallas as pl
from jax.experimental.pallas import tpu as pltpu
from jax.experimental.pallas import tpu_sc as plsc

N = 10000
E = 320000
D = 128

NC = 2
NS = 16
L = 16

NW = NC * NS
CH = 128
K = 80
E_PAD = NW * K * CH
K2 = K // 2

ROWS_PT = N // NS
OUT_PT = (N // NS) // 8 * 8
OUT_TAIL = N - OUT_PT * NS


def _combine_matmul_body(p_ref, w_ref, o_ref):
    o_ref[...] = lax.dot_general(
        p_ref[0] + p_ref[1], w_ref[...], (((1,), (1,)), ((), ())),
        preferred_element_type=jnp.float32)


def _tc_combine_matmul(partials, w):
    bm = 2000
    return pl.pallas_call(
        _combine_matmul_body,
        grid=(N // bm,),
        in_specs=[
            pl.BlockSpec((NC, bm, D), lambda i: (0, i, 0)),
            pl.BlockSpec((D, D), lambda i: (0, 0)),
        ],
        out_specs=pl.BlockSpec((bm, D), lambda i: (i, 0)),
        out_shape=jax.ShapeDtypeStruct((N, D), jnp.float32),
    )(partials, w)


def _sc_body(x_hbm, src_hbm, dst_hbm, val_hbm, out_hbm,
             src_all, dst_all, val_all, rows_a, rows_b, acc_sh,
             sem_ga, sem_gb, sem_sa, sem_sb):
    c = lax.axis_index("c")
    s = lax.axis_index("s")
    wid = c * NS + s

    zero = jnp.zeros((L,), jnp.float32)

    def z_body(i, carry):
        for r in range(D // L):
            rows_a[i, pl.ds(r * L, L)] = zero
        return carry
    lax.fori_loop(0, CH, z_body, 0)

    rbase = s * ROWS_PT
    for k in range(ROWS_PT // CH):
        pltpu.sync_copy(rows_a, acc_sh.at[pl.ds(rbase + k * CH, CH)])
    tail = ROWS_PT - (ROWS_PT // CH) * CH
    if tail:
        pltpu.sync_copy(rows_a.at[pl.ds(0, tail)],
                        acc_sh.at[pl.ds(rbase + (ROWS_PT // CH) * CH, tail)])
    plsc.subcore_barrier()

    def g_start(chunk, rows_ref, sem):
        pltpu.async_copy(x_hbm.at[src_all.at[chunk]], rows_ref, sem)

    def g_wait(rows_ref, sem):
        pltpu.make_async_copy(x_hbm.at[src_all.at[0]], rows_ref, sem).wait()

    def s_start(chunk, rows_ref, sem):
        pltpu.async_copy(rows_ref, acc_sh.at[dst_all.at[chunk]], sem, add=True)

    def s_wait(rows_ref, sem):
        pltpu.make_async_copy(rows_ref, acc_sh.at[dst_all.at[0]], sem).wait()

    def scale(rows_ref, chunk):
        def g_body(g, carry):
            vals = val_all[chunk, pl.ds(g * L, L)]
            for i in range(L):
                v = vals[i]
                e = g * L + i
                for r in range(D // L):
                    sl = pl.ds(r * L, L)
                    rows_ref[e, sl] = rows_ref[e, sl] * v
            return carry
        lax.fori_loop(0, CH // L, g_body, 0)

    P = K2 // 2
    for stage in range(K // K2):
        sbase = stage * K2
        pltpu.sync_copy(src_hbm.at[wid, pl.ds(sbase, K2)], src_all)
        pltpu.sync_copy(dst_hbm.at[wid, pl.ds(sbase, K2)], dst_all)
        pltpu.sync_copy(val_hbm.at[wid, pl.ds(sbase, K2)], val_all)

        g_start(0, rows_a, sem_ga)
        g_start(1, rows_b, sem_gb)

        def pair(p, carry):
            ca = 2 * p
            cb = ca + 1

            g_wait(rows_a, sem_ga)
            scale(rows_a, ca)
            s_start(ca, rows_a, sem_sa)

            @pl.when(p < P - 1)
            def _refill_a():
                s_wait(rows_a, sem_sa)
                g_start(ca + 2, rows_a, sem_ga)

            g_wait(rows_b, sem_gb)
            scale(rows_b, cb)
            s_start(cb, rows_b, sem_sb)

            @pl.when(p < P - 1)
            def _refill_b():
                s_wait(rows_b, sem_sb)
                g_start(cb + 2, rows_b, sem_gb)
            return carry
        lax.fori_loop(0, P, pair, 0)
        s_wait(rows_a, sem_sa)
        s_wait(rows_b, sem_sb)

    plsc.subcore_barrier()
    rows0 = s * OUT_PT
    pltpu.sync_copy(acc_sh.at[pl.ds(rows0, OUT_PT)],
                    out_hbm.at[c, pl.ds(rows0, OUT_PT)])

    @pl.when(s == 0)
    def _copy_tail():
        pltpu.sync_copy(acc_sh.at[pl.ds(OUT_PT * NS, OUT_TAIL)],
                        out_hbm.at[c, pl.ds(OUT_PT * NS, OUT_TAIL)])


_sc_aggregate = functools.partial(
    pl.kernel,
    out_type=jax.ShapeDtypeStruct((NC, N, D), jnp.float32),
    mesh=plsc.VectorSubcoreMesh(core_axis_name="c", subcore_axis_name="s"),
    scratch_types=[
        pltpu.VMEM((K2, CH), jnp.int32),
        pltpu.VMEM((K2, CH), jnp.int32),
        pltpu.VMEM((K2, CH), jnp.float32),
        pltpu.VMEM((CH, D), jnp.float32),
        pltpu.VMEM((CH, D), jnp.float32),
        pltpu.VMEM_SHARED((N, D), jnp.float32),
        pltpu.SemaphoreType.DMA,
        pltpu.SemaphoreType.DMA,
        pltpu.SemaphoreType.DMA,
        pltpu.SemaphoreType.DMA,
    ],
)(_sc_body)


def kernel(x, edge_index, edge_values, W):
    dst = edge_index[0].astype(jnp.int32)
    src = edge_index[1].astype(jnp.int32)
    pad = E_PAD - E
    spread = (jnp.arange(pad, dtype=jnp.int32) * 37) % N
    src3 = jnp.concatenate([src, spread]).reshape(NW, K, CH)
    dst3 = jnp.concatenate([dst, spread]).reshape(NW, K, CH)
    val3 = jnp.pad(edge_values, (0, pad)).reshape(NW, K, CH)
    partials = _sc_aggregate(x, src3, dst3, val3)
    return _tc_combine_matmul(partials, W)


# speedup vs baseline: 12.1973x; 1.0626x over previous
import functools

import jax
import jax.numpy as jnp
from jax import lax
from jax.experimental import pallas as pl
from jax.experimental.pallas import tpu as pltpu
from jax.experimental.pallas import tpu_sc as plsc

N = 10000
E = 320000
D = 128

NC = 2
NS = 16
L = 16

NW = NC * NS
CH = 80
K = E // (NW * CH)
STAGES = (63, 62)
KSTG = max(STAGES)

ROWS_PT = N // NS
OUT_PT = (N // NS) // 8 * 8
OUT_TAIL = N - OUT_PT * NS


def _combine_matmul_body(p_ref, w_ref, o_ref):
    o_ref[...] = lax.dot_general(
        p_ref[0] + p_ref[1], w_ref[...], (((1,), (1,)), ((), ())),
        preferred_element_type=jnp.float32)


def _tc_combine_matmul(partials, w):
    bm = 2000
    return pl.pallas_call(
        _combine_matmul_body,
        grid=(N // bm,),
        in_specs=[
            pl.BlockSpec((NC, bm, D), lambda i: (0, i, 0)),
            pl.BlockSpec((D, D), lambda i: (0, 0)),
        ],
        out_specs=pl.BlockSpec((bm, D), lambda i: (i, 0)),
        out_shape=jax.ShapeDtypeStruct((N, D), jnp.float32),
    )(partials, w)


def _sc_body(x_hbm, src_hbm, dst_hbm, val_hbm, out_hbm,
             src_all, dst_all, val_all, rows0, rows1, rows2, acc_sh,
             sem_g0, sem_g1, sem_g2, sem_s0, sem_s1, sem_s2):
    c = lax.axis_index("c")
    s = lax.axis_index("s")
    wid = c * NS + s

    rows = (rows0, rows1, rows2)
    sem_g = (sem_g0, sem_g1, sem_g2)
    sem_s = (sem_s0, sem_s1, sem_s2)

    zero = jnp.zeros((L,), jnp.float32)

    def z_body(i, carry):
        for r in range(D // L):
            rows0[i, pl.ds(r * L, L)] = zero
        return carry
    lax.fori_loop(0, CH, z_body, 0)

    rbase = s * ROWS_PT
    for k in range(ROWS_PT // CH):
        pltpu.sync_copy(rows0, acc_sh.at[pl.ds(rbase + k * CH, CH)])
    ztail = ROWS_PT - (ROWS_PT // CH) * CH
    if ztail:
        pltpu.sync_copy(rows0.at[pl.ds(0, ztail)],
                        acc_sh.at[pl.ds(rbase + (ROWS_PT // CH) * CH, ztail)])
    plsc.subcore_barrier()

    def g_start(chunk, b):
        pltpu.async_copy(x_hbm.at[src_all.at[pl.ds(chunk * CH, CH)]],
                         rows[b], sem_g[b])

    def g_wait(b):
        pltpu.make_async_copy(x_hbm.at[src_all.at[pl.ds(0, CH)]],
                              rows[b], sem_g[b]).wait()

    def s_start(chunk, b):
        pltpu.async_copy(rows[b], acc_sh.at[dst_all.at[pl.ds(chunk * CH, CH)]],
                         sem_s[b], add=True)

    def s_wait(b):
        pltpu.make_async_copy(rows[b], acc_sh.at[dst_all.at[pl.ds(0, CH)]],
                              sem_s[b]).wait()

    def scale(b, chunk):
        rows_ref = rows[b]

        def grp_body(g, carry):
            vals = val_all[pl.ds(chunk * CH + g * L, L)]
            for i in range(L):
                v = vals[i]
                e = g * L + i
                for r in range(D // L):
                    sl = pl.ds(r * L, L)
                    rows_ref[e, sl] = rows_ref[e, sl] * v
            return carry
        lax.fori_loop(0, CH // L, grp_body, 0)

    def emit_chunk(ch, b, first, do_gather):
        g_wait(b)
        scale(b, ch)
        s_start(ch, b)
        yb = (b + 2) % 3
        if not first:
            s_wait(yb)
        if do_gather:
            g_start(ch + 2, yb)

    ebase = wid * (K * CH)
    sbase = 0
    for S in STAGES:
        pltpu.sync_copy(src_hbm.at[pl.ds(ebase + sbase * CH, S * CH)],
                        src_all.at[pl.ds(0, S * CH)])
        pltpu.sync_copy(dst_hbm.at[pl.ds(ebase + sbase * CH, S * CH)],
                        dst_all.at[pl.ds(0, S * CH)])
        pltpu.sync_copy(val_hbm.at[pl.ds(ebase + sbase * CH, S * CH)],
                        val_all.at[pl.ds(0, S * CH)])

        g_start(0, 0)
        g_start(1, 1)
        emit_chunk(0, 0, first=True, do_gather=True)

        T = (S - 3) // 3

        def triple(t, carry):
            ch = 3 * t + 1
            emit_chunk(ch, 1, first=False, do_gather=True)
            emit_chunk(ch + 1, 2, first=False, do_gather=True)
            emit_chunk(ch + 2, 0, first=False, do_gather=True)
            return carry
        lax.fori_loop(0, T, triple, 0)

        for ch in range(3 * T + 1, S):
            emit_chunk(ch, ch % 3, first=False, do_gather=(ch + 2 < S))
        s_wait((S - 1) % 3)
        sbase += S

    plsc.subcore_barrier()
    orow = s * OUT_PT
    pltpu.sync_copy(acc_sh.at[pl.ds(orow, OUT_PT)],
                    out_hbm.at[c, pl.ds(orow, OUT_PT)])

    @pl.when(s == 0)
    def _copy_tail():
        pltpu.sync_copy(acc_sh.at[pl.ds(OUT_PT * NS, OUT_TAIL)],
                        out_hbm.at[c, pl.ds(OUT_PT * NS, OUT_TAIL)])


_sc_aggregate = functools.partial(
    pl.kernel,
    out_type=jax.ShapeDtypeStruct((NC, N, D), jnp.float32),
    mesh=plsc.VectorSubcoreMesh(core_axis_name="c", subcore_axis_name="s"),
    scratch_types=[
        pltpu.VMEM((KSTG * CH,), jnp.int32),
        pltpu.VMEM((KSTG * CH,), jnp.int32),
        pltpu.VMEM((KSTG * CH,), jnp.float32),
        pltpu.VMEM((CH, D), jnp.float32),
        pltpu.VMEM((CH, D), jnp.float32),
        pltpu.VMEM((CH, D), jnp.float32),
        pltpu.VMEM_SHARED((N, D), jnp.float32),
        pltpu.SemaphoreType.DMA,
        pltpu.SemaphoreType.DMA,
        pltpu.SemaphoreType.DMA,
        pltpu.SemaphoreType.DMA,
        pltpu.SemaphoreType.DMA,
        pltpu.SemaphoreType.DMA,
    ],
)(_sc_body)


def kernel(x, edge_index, edge_values, W):
    dst = edge_index[0].astype(jnp.int32)
    src = edge_index[1].astype(jnp.int32)
    partials = _sc_aggregate(x, src, dst, edge_values)
    return _tc_combine_matmul(partials, W)
